# Initial kernel scaffold; baseline (speedup 1.0000x reference)
import jax
import jax.numpy as jnp
from jax.experimental import pallas as pl


def kernel(image, post_tran, post_rot, intrinsic, extrinsic, W_enc, b_enc):
    raise NotImplementedError("write your pallas kernel here")


# speedup vs baseline: 1.5619x; 1.5619x over previous
import jax
import jax.numpy as jnp
from jax import lax
from jax.experimental import pallas as pl
from jax.experimental.pallas import tpu as pltpu

B, N, H, W = 2, 6, 256, 704
DS = 16
DSH, DSW = H // DS, W // DS
D, C_FEAT = 60, 64
X_DIM, Y_DIM, Z_DIM = 200, 200, 1
HW = DSH * DSW
BN = B * N
NSEG = X_DIM * Y_DIM


def _frustum():
    depth = jnp.arange(1.0, 61.0, 1.0, dtype=jnp.float32)
    depth = jnp.broadcast_to(depth[:, None, None], (D, DSH, DSW))
    xg = jnp.linspace(0.0, W - 1.0, DSW, dtype=jnp.float32)
    xg = jnp.broadcast_to(xg[None, None, :], (D, DSH, DSW))
    yg = jnp.linspace(0.0, H - 1.0, DSH, dtype=jnp.float32)
    yg = jnp.broadcast_to(yg[None, :, None], (D, DSH, DSW))
    return jnp.stack((xg, yg, depth), -1)


def _enc_body(img_ref, wenc_ref, benc_ref, geom_ref,
              depth_ref, feat_ref, w_ref, rank_ref):
    img = img_ref[0]

    rHt = lax.broadcasted_iota(jnp.int32, (DSH, H), 0)
    cHt = lax.broadcasted_iota(jnp.int32, (DSH, H), 1) // DS
    PhT = jnp.where(rHt == cHt, 1.0 / DS, 0.0).astype(jnp.float32)
    rW = lax.broadcasted_iota(jnp.int32, (W, DSW), 0) // DS
    cW = lax.broadcasted_iota(jnp.int32, (W, DSW), 1)
    Pw = jnp.where(rW == cW, 1.0 / DS, 0.0).astype(jnp.float32)

    a = jnp.dot(img.reshape(3 * H, W), Pw,
                preferred_element_type=jnp.float32)
    pooled_rows = []
    for c in range(3):
        ac = a[c * H:(c + 1) * H, :]
        pc = jnp.dot(PhT, ac, preferred_element_type=jnp.float32)
        pooled_rows.append(pc)
    pooled = jnp.stack(pooled_rows, axis=0)

    wenc = wenc_ref[...]
    logits = (wenc[:, 0][:, None, None] * pooled[0][None]
              + wenc[:, 1][:, None, None] * pooled[1][None]
              + wenc[:, 2][:, None, None] * pooled[2][None])
    benc = benc_ref[...]
    logits = logits + benc[:, :, None]

    ls = logits[:D]
    m = jnp.max(ls, axis=0, keepdims=True)
    e = jnp.exp(ls - m)
    s = jnp.sum(e, axis=0, keepdims=True)
    depth = e / s
    feat = logits[D:]

    g = geom_ref[0]
    vx = ((g[0] - (-50.25)) / 0.5).astype(jnp.int32)
    vy = ((g[1] - (-50.25)) / 0.5).astype(jnp.int32)
    vz = ((g[2] - (-20.0)) / 20.0).astype(jnp.int32)
    mask = ((vx >= 0) & (vx < X_DIM) & (vy >= 0) & (vy < Y_DIM)
            & (vz >= 0) & (vz < Z_DIM))
    rank = vx * (Y_DIM * Z_DIM) + vy * Z_DIM + vz
    rank = jnp.where(mask, rank, 0)

    depth_ref[0] = depth
    feat_ref[0] = feat
    w_ref[0] = depth * mask.astype(jnp.float32)
    rank_ref[0] = rank


def _encode(image, W_enc, b_enc, geom_t):
    img = image.reshape(BN, 3, H, W)
    benc2 = b_enc.reshape(D + C_FEAT, 1)
    return pl.pallas_call(
        _enc_body,
        grid=(BN,),
        in_specs=[
            pl.BlockSpec((1, 3, H, W), lambda i: (i, 0, 0, 0)),
            pl.BlockSpec((D + C_FEAT, 3), lambda i: (0, 0)),
            pl.BlockSpec((D + C_FEAT, 1), lambda i: (0, 0)),
            pl.BlockSpec((1, 3, D, DSH, DSW), lambda i: (i, 0, 0, 0, 0)),
        ],
        out_specs=[
            pl.BlockSpec((1, D, DSH, DSW), lambda i: (i, 0, 0, 0)),
            pl.BlockSpec((1, C_FEAT, DSH, DSW), lambda i: (i, 0, 0, 0)),
            pl.BlockSpec((1, D, DSH, DSW), lambda i: (i, 0, 0, 0)),
            pl.BlockSpec((1, D, DSH, DSW), lambda i: (i, 0, 0, 0)),
        ],
        out_shape=[
            jax.ShapeDtypeStruct((BN, D, DSH, DSW), jnp.float32),
            jax.ShapeDtypeStruct((BN, C_FEAT, DSH, DSW), jnp.float32),
            jax.ShapeDtypeStruct((BN, D, DSH, DSW), jnp.float32),
            jax.ShapeDtypeStruct((BN, D, DSH, DSW), jnp.int32),
        ],
    )(img, W_enc, benc2, geom_t)


def kernel(image, post_tran, post_rot, intrinsic, extrinsic, W_enc, b_enc):
    frustum = _frustum()
    ext_inv = jnp.linalg.inv(extrinsic)
    rotation = ext_inv[..., :3, :3]
    translation = ext_inv[..., :3, 3]
    points = frustum[None, None] - post_tran[:, :, None, None, None, :]
    points = jnp.einsum('bnij,bndhwj->bndhwi', jnp.linalg.inv(post_rot), points)
    points = jnp.concatenate((points[..., :2] * points[..., 2:3], points[..., 2:3]), axis=-1)
    combine = jnp.einsum('bnij,bnjk->bnik', rotation, jnp.linalg.inv(intrinsic))
    points = jnp.einsum('bnij,bndhwj->bndhwi', combine, points)
    geom = points + translation[:, :, None, None, None, :]

    geom_t = geom.reshape(BN, D, DSH, DSW, 3).transpose(0, 4, 1, 2, 3)

    depth, feat, w, rank = _encode(image, W_enc, b_enc, geom_t)

    depth_out = depth.reshape(B, N, D, DSH, DSW)

    vals = (w.reshape(B, N, D, HW)[..., None]
            * feat.reshape(B, N, C_FEAT, HW).transpose(0, 1, 3, 2)[:, :, None])
    vals = vals.reshape(B, N * D * HW, C_FEAT)
    ranks = rank.reshape(B, N * D * HW)
    outs = []
    for b in range(B):
        flat = jax.ops.segment_sum(vals[b], ranks[b], num_segments=NSEG)
        outs.append(flat.reshape(X_DIM, Y_DIM, C_FEAT).transpose(2, 0, 1))
    output = jnp.stack(outs, axis=0)
    return output, depth_out


# speedup vs baseline: 2.1987x; 1.4077x over previous
import functools

import jax
import jax.numpy as jnp
from jax import lax
from jax.experimental import pallas as pl
from jax.experimental.pallas import tpu as pltpu
from jax.experimental.pallas import tpu_sc as plsc

B, N, H, W = 2, 6, 256, 704
DS = 16
DSH, DSW = H // DS, W // DS
D, C_FEAT = 60, 64
X_DIM, Y_DIM, Z_DIM = 200, 200, 1
HW = DSH * DSW
BN = B * N
NSEG = X_DIM * Y_DIM


def _frustum():
    depth = jnp.arange(1.0, 61.0, 1.0, dtype=jnp.float32)
    depth = jnp.broadcast_to(depth[:, None, None], (D, DSH, DSW))
    xg = jnp.linspace(0.0, W - 1.0, DSW, dtype=jnp.float32)
    xg = jnp.broadcast_to(xg[None, None, :], (D, DSH, DSW))
    yg = jnp.linspace(0.0, H - 1.0, DSH, dtype=jnp.float32)
    yg = jnp.broadcast_to(yg[None, :, None], (D, DSH, DSW))
    return jnp.stack((xg, yg, depth), -1)


def _enc_body(img_ref, wenc_ref, benc_ref, geom_ref,
              depth_ref, feat_ref, w_ref, rank_ref):
    img = img_ref[0]

    rHt = lax.broadcasted_iota(jnp.int32, (DSH, H), 0)
    cHt = lax.broadcasted_iota(jnp.int32, (DSH, H), 1) // DS
    PhT = jnp.where(rHt == cHt, 1.0 / DS, 0.0).astype(jnp.float32)
    rW = lax.broadcasted_iota(jnp.int32, (W, DSW), 0) // DS
    cW = lax.broadcasted_iota(jnp.int32, (W, DSW), 1)
    Pw = jnp.where(rW == cW, 1.0 / DS, 0.0).astype(jnp.float32)

    a = jnp.dot(img.reshape(3 * H, W), Pw,
                preferred_element_type=jnp.float32)
    pooled_rows = []
    for c in range(3):
        ac = a[c * H:(c + 1) * H, :]
        pc = jnp.dot(PhT, ac, preferred_element_type=jnp.float32)
        pooled_rows.append(pc)
    pooled = jnp.stack(pooled_rows, axis=0)

    wenc = wenc_ref[...]
    logits = (wenc[:, 0][:, None, None] * pooled[0][None]
              + wenc[:, 1][:, None, None] * pooled[1][None]
              + wenc[:, 2][:, None, None] * pooled[2][None])
    benc = benc_ref[...]
    logits = logits + benc[:, :, None]

    ls = logits[:D]
    m = jnp.max(ls, axis=0, keepdims=True)
    e = jnp.exp(ls - m)
    s = jnp.sum(e, axis=0, keepdims=True)
    depth = e / s
    feat = logits[D:]

    g = geom_ref[0]
    vx = ((g[0] - (-50.25)) / 0.5).astype(jnp.int32)
    vy = ((g[1] - (-50.25)) / 0.5).astype(jnp.int32)
    vz = ((g[2] - (-20.0)) / 20.0).astype(jnp.int32)
    mask = ((vx >= 0) & (vx < X_DIM) & (vy >= 0) & (vy < Y_DIM)
            & (vz >= 0) & (vz < Z_DIM))
    rank = vx * (Y_DIM * Z_DIM) + vy * Z_DIM + vz
    rank = jnp.where(mask, rank, 0)

    depth_ref[0] = depth
    feat_ref[0] = feat
    w_ref[0] = depth * mask.astype(jnp.float32)
    rank_ref[0] = rank


def _encode(image, W_enc, b_enc, geom_t):
    img = image.reshape(BN, 3, H, W)
    benc2 = b_enc.reshape(D + C_FEAT, 1)
    return pl.pallas_call(
        _enc_body,
        grid=(BN,),
        in_specs=[
            pl.BlockSpec((1, 3, H, W), lambda i: (i, 0, 0, 0)),
            pl.BlockSpec((D + C_FEAT, 3), lambda i: (0, 0)),
            pl.BlockSpec((D + C_FEAT, 1), lambda i: (0, 0)),
            pl.BlockSpec((1, 3, D, DSH, DSW), lambda i: (i, 0, 0, 0, 0)),
        ],
        out_specs=[
            pl.BlockSpec((1, D, DSH, DSW), lambda i: (i, 0, 0, 0)),
            pl.BlockSpec((1, C_FEAT, DSH, DSW), lambda i: (i, 0, 0, 0)),
            pl.BlockSpec((1, D, DSH, DSW), lambda i: (i, 0, 0, 0)),
            pl.BlockSpec((1, D, DSH, DSW), lambda i: (i, 0, 0, 0)),
        ],
        out_shape=[
            jax.ShapeDtypeStruct((BN, D, DSH, DSW), jnp.float32),
            jax.ShapeDtypeStruct((BN, C_FEAT, DSH, DSW), jnp.float32),
            jax.ShapeDtypeStruct((BN, D, DSH, DSW), jnp.float32),
            jax.ShapeDtypeStruct((BN, D, DSH, DSW), jnp.int32),
        ],
    )(img, W_enc, benc2, geom_t)


NTILE = 16
PIX_T = HW * N // NTILE
PTS_T_RAW = PIX_T * D
BLKP = 128
NBLK = (PTS_T_RAW + BLKP - 1) // BLKP
NPASS = 4
CH = C_FEAT // NPASS
ROWS_T = 2512
VROWS = NTILE * ROWS_T
OUT_ROWS = 320

ROWS_PTS = 128
PTS_PAD = ROWS_PTS * 128
OFF_RW = 0
OFF_PIX = OFF_RW + B * NTILE * ROWS_PTS
ROWS_FPT = 40
OFF_FEAT = OFF_PIX + ROWS_PTS
BLOB_ROWS = OFF_FEAT + B * NPASS * NTILE * ROWS_FPT
W_DESCALE = 1.0 / (65535.0 * float(2 ** 20))


def _sc_scatter_body(blob_hbm, zeros_hbm, out_hbm,
                     feat_v, rw_v, pix_v, idx_v, rows_v,
                     grid_sh):
    b = lax.axis_index("c")
    s = lax.axis_index("s")

    pltpu.sync_copy(blob_hbm.at[pl.ds(OFF_PIX, ROWS_PTS)], pix_v)
    pltpu.sync_copy(blob_hbm.at[pl.ds(OFF_RW + (b * NTILE + s) * ROWS_PTS,
                                      ROWS_PTS)], rw_v)

    def pass_body(p, _):
        pltpu.sync_copy(zeros_hbm, grid_sh.at[pl.ds(s * ROWS_T, ROWS_T)])
        pltpu.sync_copy(
            blob_hbm.at[pl.ds(OFF_FEAT + ((b * NPASS + p) * NTILE + s)
                              * ROWS_FPT, ROWS_FPT)], feat_v)
        plsc.subcore_barrier()

        def blk_body(blk, _):
            def grp_body(g, _):
                q16 = g * 16
                pvec = rw_v[blk, pl.ds(q16, 16)]
                idx_v[0, pl.ds(q16, 16)] = lax.shift_right_logical(pvec, 16)
                pixvec = pix_v[blk, pl.ds(q16, 16)]
                for l in range(16):
                    wl = ((pvec[l] & 0xFFFF).astype(jnp.float32) * W_DESCALE)
                    pixl = pixvec[l]
                    fr = lax.shift_right_logical(pixl, 3)
                    fc = (pixl & 7) * CH
                    fvec = feat_v[fr, pl.ds(fc, 16)].astype(jnp.float32)
                    rows_v[g * 16 + l, 0:16] = wl * fvec
                return 0
            lax.fori_loop(0, BLKP // 16, grp_body, 0)
            pltpu.sync_copy(rows_v, grid_sh.at[idx_v.at[0]], add=True)
            return 0
        lax.fori_loop(0, NBLK, blk_body, 0)
        plsc.subcore_barrier()

        pltpu.sync_copy(grid_sh.at[pl.ds(s * ROWS_T, ROWS_T)],
                        out_hbm.at[b, p, pl.ds(s * ROWS_T, ROWS_T)])
        plsc.subcore_barrier()
        return 0

    lax.fori_loop(0, NPASS, pass_body, 0)


def _sc_scatter(blob):
    mesh = plsc.VectorSubcoreMesh(core_axis_name="c", subcore_axis_name="s")
    f = pl.kernel(
        _sc_scatter_body, mesh=mesh,
        compiler_params=pltpu.CompilerParams(use_tc_tiling_on_sc=False),
        out_type=jax.ShapeDtypeStruct((B, NPASS, VROWS, CH), jnp.float32),
        scratch_types=[
            pltpu.VMEM((ROWS_FPT, 128), jnp.int32),
            pltpu.VMEM((ROWS_PTS, 128), jnp.int32),
            pltpu.VMEM((ROWS_PTS, 128), jnp.int32),
            pltpu.VMEM((1, 128), jnp.int32),
            pltpu.VMEM((BLKP, CH), jnp.float32),
            pltpu.VMEM_SHARED((VROWS, CH), jnp.float32),
        ],
    )
    return f(blob, jnp.zeros((ROWS_T, CH), jnp.float32))


def kernel(image, post_tran, post_rot, intrinsic, extrinsic, W_enc, b_enc):
    frustum = _frustum()
    ext_inv = jnp.linalg.inv(extrinsic)
    rotation = ext_inv[..., :3, :3]
    translation = ext_inv[..., :3, 3]
    points = frustum[None, None] - post_tran[:, :, None, None, None, :]
    points = jnp.einsum('bnij,bndhwj->bndhwi', jnp.linalg.inv(post_rot), points)
    points = jnp.concatenate((points[..., :2] * points[..., 2:3], points[..., 2:3]), axis=-1)
    combine = jnp.einsum('bnij,bnjk->bnik', rotation, jnp.linalg.inv(intrinsic))
    points = jnp.einsum('bnij,bndhwj->bndhwi', combine, points)
    geom = points + translation[:, :, None, None, None, :]

    geom_t = geom.reshape(BN, D, DSH, DSW, 3).transpose(0, 4, 1, 2, 3)

    depth, feat, w, rank = _encode(image, W_enc, b_enc, geom_t)

    depth_out = depth.reshape(B, N, D, DSH, DSW)

    npix = N * HW
    w_pm = w.reshape(B, N, D, HW).transpose(0, 1, 3, 2).reshape(B, npix * D)
    r_pm = rank.reshape(B, N, D, HW).transpose(0, 1, 3, 2).reshape(B, npix * D)
    w16 = jnp.round(w_pm * 65535.0).astype(jnp.uint32)
    packed = (r_pm.astype(jnp.uint32) << 16) | w16
    packed = jnp.pad(packed.reshape(B, NTILE, PTS_T_RAW),
                     ((0, 0), (0, 0), (0, PTS_PAD - PTS_T_RAW)),
                     constant_values=jnp.uint32((VROWS - 1) << 16))
    pix_map = jnp.minimum(jnp.arange(PTS_PAD, dtype=jnp.int32) // D, PIX_T - 1)
    feat_sc = (feat.reshape(B, N, C_FEAT, HW).transpose(0, 1, 3, 2)
               .reshape(B, npix, NPASS, CH).transpose(0, 2, 1, 3)
               .reshape(B, NPASS, NTILE, PIX_T * CH))
    feat_sc = jnp.pad(feat_sc,
                      ((0, 0), (0, 0), (0, 0), (0, ROWS_FPT * 128 - PIX_T * CH)))

    blob = jnp.concatenate([
        lax.bitcast_convert_type(packed, jnp.int32).reshape(-1),
        pix_map,
        (feat_sc * (2.0 ** 20)).astype(jnp.int32).reshape(-1),
    ]).reshape(BLOB_ROWS, 128)

    grid = _sc_scatter(blob)
    flat = grid.transpose(0, 2, 1, 3).reshape(B, VROWS, C_FEAT)[:, :NSEG]
    output = flat.reshape(B, X_DIM, Y_DIM, C_FEAT).transpose(0, 3, 1, 2)
    return output, depth_out


# speedup vs baseline: 3.4039x; 1.5481x over previous
import functools

import jax
import jax.numpy as jnp
from jax import lax
from jax.experimental import pallas as pl
from jax.experimental.pallas import tpu as pltpu
from jax.experimental.pallas import tpu_sc as plsc

B, N, H, W = 2, 6, 256, 704
DS = 16
DSH, DSW = H // DS, W // DS
D, C_FEAT = 60, 64
X_DIM, Y_DIM, Z_DIM = 200, 200, 1
HW = DSH * DSW
BN = B * N
NSEG = X_DIM * Y_DIM


def _frustum():
    depth = jnp.arange(1.0, 61.0, 1.0, dtype=jnp.float32)
    depth = jnp.broadcast_to(depth[:, None, None], (D, DSH, DSW))
    xg = jnp.linspace(0.0, W - 1.0, DSW, dtype=jnp.float32)
    xg = jnp.broadcast_to(xg[None, None, :], (D, DSH, DSW))
    yg = jnp.linspace(0.0, H - 1.0, DSH, dtype=jnp.float32)
    yg = jnp.broadcast_to(yg[None, :, None], (D, DSH, DSW))
    return jnp.stack((xg, yg, depth), -1)


def _enc_body(img_ref, wenc_ref, benc_ref, geom_ref,
              depth_ref, feat_ref, w_ref, rank_ref):
    img = img_ref[0]

    rHt = lax.broadcasted_iota(jnp.int32, (DSH, H), 0)
    cHt = lax.broadcasted_iota(jnp.int32, (DSH, H), 1) // DS
    PhT = jnp.where(rHt == cHt, 1.0 / DS, 0.0).astype(jnp.float32)
    rW = lax.broadcasted_iota(jnp.int32, (W, DSW), 0) // DS
    cW = lax.broadcasted_iota(jnp.int32, (W, DSW), 1)
    Pw = jnp.where(rW == cW, 1.0 / DS, 0.0).astype(jnp.float32)

    a = jnp.dot(img.reshape(3 * H, W), Pw,
                preferred_element_type=jnp.float32)
    pooled_rows = []
    for c in range(3):
        ac = a[c * H:(c + 1) * H, :]
        pc = jnp.dot(PhT, ac, preferred_element_type=jnp.float32)
        pooled_rows.append(pc)
    pooled = jnp.stack(pooled_rows, axis=0)

    wenc = wenc_ref[...]
    logits = (wenc[:, 0][:, None, None] * pooled[0][None]
              + wenc[:, 1][:, None, None] * pooled[1][None]
              + wenc[:, 2][:, None, None] * pooled[2][None])
    benc = benc_ref[...]
    logits = logits + benc[:, :, None]

    ls = logits[:D]
    m = jnp.max(ls, axis=0, keepdims=True)
    e = jnp.exp(ls - m)
    s = jnp.sum(e, axis=0, keepdims=True)
    depth = e / s
    feat = logits[D:]

    g = geom_ref[0]
    vx = ((g[0] - (-50.25)) / 0.5).astype(jnp.int32)
    vy = ((g[1] - (-50.25)) / 0.5).astype(jnp.int32)
    vz = ((g[2] - (-20.0)) / 20.0).astype(jnp.int32)
    mask = ((vx >= 0) & (vx < X_DIM) & (vy >= 0) & (vy < Y_DIM)
            & (vz >= 0) & (vz < Z_DIM))
    rank = vx * (Y_DIM * Z_DIM) + vy * Z_DIM + vz
    rank = jnp.where(mask, rank, 0)

    depth_ref[0] = depth
    feat_ref[0] = feat
    w_ref[0] = depth * mask.astype(jnp.float32)
    rank_ref[0] = rank


def _encode(image, W_enc, b_enc, geom_t):
    img = image.reshape(BN, 3, H, W)
    benc2 = b_enc.reshape(D + C_FEAT, 1)
    return pl.pallas_call(
        _enc_body,
        grid=(BN,),
        in_specs=[
            pl.BlockSpec((1, 3, H, W), lambda i: (i, 0, 0, 0)),
            pl.BlockSpec((D + C_FEAT, 3), lambda i: (0, 0)),
            pl.BlockSpec((D + C_FEAT, 1), lambda i: (0, 0)),
            pl.BlockSpec((1, 3, D, DSH, DSW), lambda i: (i, 0, 0, 0, 0)),
        ],
        out_specs=[
            pl.BlockSpec((1, D, DSH, DSW), lambda i: (i, 0, 0, 0)),
            pl.BlockSpec((1, C_FEAT, DSH, DSW), lambda i: (i, 0, 0, 0)),
            pl.BlockSpec((1, D, DSH, DSW), lambda i: (i, 0, 0, 0)),
            pl.BlockSpec((1, D, DSH, DSW), lambda i: (i, 0, 0, 0)),
        ],
        out_shape=[
            jax.ShapeDtypeStruct((BN, D, DSH, DSW), jnp.float32),
            jax.ShapeDtypeStruct((BN, C_FEAT, DSH, DSW), jnp.float32),
            jax.ShapeDtypeStruct((BN, D, DSH, DSW), jnp.float32),
            jax.ShapeDtypeStruct((BN, D, DSH, DSW), jnp.int32),
        ],
    )(img, W_enc, benc2, geom_t)


NTILE = 16
PIX_T = HW * N // NTILE
PTS_T_RAW = PIX_T * D
BLKP = 128
NBLK = (PTS_T_RAW + BLKP - 1) // BLKP
NPASS = 2
CH = C_FEAT // NPASS
ROWS_T = 2512
VROWS = NTILE * ROWS_T
OUT_ROWS = 320

ROWS_PTS = 128
PTS_PAD = ROWS_PTS * 128
OFF_RW = 0
OFF_PIX = OFF_RW + B * NTILE * ROWS_PTS
ROWS_FPT = 72
OFF_FEAT = OFF_PIX + ROWS_PTS
BLOB_ROWS = OFF_FEAT + B * NPASS * NTILE * ROWS_FPT
W_DESCALE = 1.0 / (65535.0 * float(2 ** 20))


def _sc_scatter_body(blob_hbm, zeros_hbm, out_hbm,
                     feat_v, rw_v, pix_v, idx_v, rows_v,
                     grid_sh):
    b = lax.axis_index("c")
    s = lax.axis_index("s")

    pltpu.sync_copy(blob_hbm.at[pl.ds(OFF_PIX, ROWS_PTS)], pix_v)
    pltpu.sync_copy(blob_hbm.at[pl.ds(OFF_RW + (b * NTILE + s) * ROWS_PTS,
                                      ROWS_PTS)], rw_v)

    def pass_body(p, _):
        pltpu.sync_copy(zeros_hbm, grid_sh.at[pl.ds(s * ROWS_T, ROWS_T)])
        pltpu.sync_copy(
            blob_hbm.at[pl.ds(OFF_FEAT + ((b * NPASS + p) * NTILE + s)
                              * ROWS_FPT, ROWS_FPT)], feat_v)
        plsc.subcore_barrier()

        def blk_body(blk, _):
            def grp_body(g, _):
                q16 = g * 16
                pvec = rw_v[blk, pl.ds(q16, 16)]
                idx_v[0, pl.ds(q16, 16)] = lax.shift_right_logical(pvec, 16)
                pixvec = pix_v[blk, pl.ds(q16, 16)]
                for l in range(16):
                    wl = ((pvec[l] & 0xFFFF).astype(jnp.float32) * W_DESCALE)
                    pixl = pixvec[l]
                    fr = lax.shift_right_logical(pixl, 2)
                    fc = (pixl & 3) * CH
                    for j in range(CH // 16):
                        fvec = (feat_v[fr, pl.ds(fc + 16 * j, 16)]
                                .astype(jnp.float32))
                        rows_v[g * 16 + l, 16 * j:16 * (j + 1)] = wl * fvec
                return 0
            lax.fori_loop(0, BLKP // 16, grp_body, 0)
            pltpu.sync_copy(rows_v, grid_sh.at[idx_v.at[0]], add=True)
            return 0
        lax.fori_loop(0, NBLK, blk_body, 0)
        plsc.subcore_barrier()

        pltpu.sync_copy(grid_sh.at[pl.ds(s * ROWS_T, ROWS_T)],
                        out_hbm.at[b, p, pl.ds(s * ROWS_T, ROWS_T)])
        plsc.subcore_barrier()
        return 0

    lax.fori_loop(0, NPASS, pass_body, 0)


def _sc_scatter(blob):
    mesh = plsc.VectorSubcoreMesh(core_axis_name="c", subcore_axis_name="s")
    f = pl.kernel(
        _sc_scatter_body, mesh=mesh,
        compiler_params=pltpu.CompilerParams(use_tc_tiling_on_sc=False),
        out_type=jax.ShapeDtypeStruct((B, NPASS, VROWS, CH), jnp.float32),
        scratch_types=[
            pltpu.VMEM((ROWS_FPT, 128), jnp.int32),
            pltpu.VMEM((ROWS_PTS, 128), jnp.int32),
            pltpu.VMEM((ROWS_PTS, 128), jnp.int32),
            pltpu.VMEM((1, 128), jnp.int32),
            pltpu.VMEM((BLKP, CH), jnp.float32),
            pltpu.VMEM_SHARED((VROWS, CH), jnp.float32),
        ],
    )
    return f(blob, jnp.zeros((ROWS_T, CH), jnp.float32))


def kernel(image, post_tran, post_rot, intrinsic, extrinsic, W_enc, b_enc):
    frustum = _frustum()
    ext_inv = jnp.linalg.inv(extrinsic)
    rotation = ext_inv[..., :3, :3]
    translation = ext_inv[..., :3, 3]
    points = frustum[None, None] - post_tran[:, :, None, None, None, :]
    points = jnp.einsum('bnij,bndhwj->bndhwi', jnp.linalg.inv(post_rot), points)
    points = jnp.concatenate((points[..., :2] * points[..., 2:3], points[..., 2:3]), axis=-1)
    combine = jnp.einsum('bnij,bnjk->bnik', rotation, jnp.linalg.inv(intrinsic))
    points = jnp.einsum('bnij,bndhwj->bndhwi', combine, points)
    geom = points + translation[:, :, None, None, None, :]

    geom_t = geom.reshape(BN, D, DSH, DSW, 3).transpose(0, 4, 1, 2, 3)

    depth, feat, w, rank = _encode(image, W_enc, b_enc, geom_t)

    depth_out = depth.reshape(B, N, D, DSH, DSW)

    npix = N * HW
    w_pm = w.reshape(B, N, D, HW).transpose(0, 1, 3, 2).reshape(B, npix * D)
    r_pm = rank.reshape(B, N, D, HW).transpose(0, 1, 3, 2).reshape(B, npix * D)
    w16 = jnp.round(w_pm * 65535.0).astype(jnp.uint32)
    packed = (r_pm.astype(jnp.uint32) << 16) | w16
    packed = jnp.pad(packed.reshape(B, NTILE, PTS_T_RAW),
                     ((0, 0), (0, 0), (0, PTS_PAD - PTS_T_RAW)),
                     constant_values=jnp.uint32((VROWS - 1) << 16))
    pix_map = jnp.minimum(jnp.arange(PTS_PAD, dtype=jnp.int32) // D, PIX_T - 1)
    feat_sc = (feat.reshape(B, N, C_FEAT, HW).transpose(0, 1, 3, 2)
               .reshape(B, npix, NPASS, CH).transpose(0, 2, 1, 3)
               .reshape(B, NPASS, NTILE, PIX_T * CH))
    feat_sc = jnp.pad(feat_sc,
                      ((0, 0), (0, 0), (0, 0), (0, ROWS_FPT * 128 - PIX_T * CH)))

    blob = jnp.concatenate([
        lax.bitcast_convert_type(packed, jnp.int32).reshape(-1),
        pix_map,
        (feat_sc * (2.0 ** 20)).astype(jnp.int32).reshape(-1),
    ]).reshape(BLOB_ROWS, 128)

    grid = _sc_scatter(blob)
    flat = grid.transpose(0, 2, 1, 3).reshape(B, VROWS, C_FEAT)[:, :NSEG]
    output = flat.reshape(B, X_DIM, Y_DIM, C_FEAT).transpose(0, 3, 1, 2)
    return output, depth_out


# speedup vs baseline: 3.4056x; 1.0005x over previous
import functools

import jax
import jax.numpy as jnp
from jax import lax
from jax.experimental import pallas as pl
from jax.experimental.pallas import tpu as pltpu
from jax.experimental.pallas import tpu_sc as plsc

B, N, H, W = 2, 6, 256, 704
DS = 16
DSH, DSW = H // DS, W // DS
D, C_FEAT = 60, 64
X_DIM, Y_DIM, Z_DIM = 200, 200, 1
HW = DSH * DSW
BN = B * N
NSEG = X_DIM * Y_DIM


def _frustum():
    depth = jnp.arange(1.0, 61.0, 1.0, dtype=jnp.float32)
    depth = jnp.broadcast_to(depth[:, None, None], (D, DSH, DSW))
    xg = jnp.linspace(0.0, W - 1.0, DSW, dtype=jnp.float32)
    xg = jnp.broadcast_to(xg[None, None, :], (D, DSH, DSW))
    yg = jnp.linspace(0.0, H - 1.0, DSH, dtype=jnp.float32)
    yg = jnp.broadcast_to(yg[None, :, None], (D, DSH, DSW))
    return jnp.stack((xg, yg, depth), -1)


def _enc_body(img_ref, wenc_ref, benc_ref, geom_ref,
              depth_ref, feat_ref, w_ref, rank_ref):
    img = img_ref[0]

    rHt = lax.broadcasted_iota(jnp.int32, (DSH, H), 0)
    cHt = lax.broadcasted_iota(jnp.int32, (DSH, H), 1) // DS
    PhT = jnp.where(rHt == cHt, 1.0 / DS, 0.0).astype(jnp.float32)
    rW = lax.broadcasted_iota(jnp.int32, (W, DSW), 0) // DS
    cW = lax.broadcasted_iota(jnp.int32, (W, DSW), 1)
    Pw = jnp.where(rW == cW, 1.0 / DS, 0.0).astype(jnp.float32)

    a = jnp.dot(img.reshape(3 * H, W), Pw,
                preferred_element_type=jnp.float32)
    pooled_rows = []
    for c in range(3):
        ac = a[c * H:(c + 1) * H, :]
        pc = jnp.dot(PhT, ac, preferred_element_type=jnp.float32)
        pooled_rows.append(pc)
    pooled = jnp.stack(pooled_rows, axis=0)

    wenc = wenc_ref[...]
    logits = (wenc[:, 0][:, None, None] * pooled[0][None]
              + wenc[:, 1][:, None, None] * pooled[1][None]
              + wenc[:, 2][:, None, None] * pooled[2][None])
    benc = benc_ref[...]
    logits = logits + benc[:, :, None]

    ls = logits[:D]
    m = jnp.max(ls, axis=0, keepdims=True)
    e = jnp.exp(ls - m)
    s = jnp.sum(e, axis=0, keepdims=True)
    depth = e / s
    feat = logits[D:]

    g = geom_ref[0]
    vx = ((g[0] - (-50.25)) / 0.5).astype(jnp.int32)
    vy = ((g[1] - (-50.25)) / 0.5).astype(jnp.int32)
    vz = ((g[2] - (-20.0)) / 20.0).astype(jnp.int32)
    mask = ((vx >= 0) & (vx < X_DIM) & (vy >= 0) & (vy < Y_DIM)
            & (vz >= 0) & (vz < Z_DIM))
    rank = vx * (Y_DIM * Z_DIM) + vy * Z_DIM + vz
    rank = jnp.where(mask, rank, 0)

    depth_ref[0] = depth
    feat_ref[0] = feat
    w_ref[0] = depth * mask.astype(jnp.float32)
    rank_ref[0] = rank


def _encode(image, W_enc, b_enc, geom_t):
    img = image.reshape(BN, 3, H, W)
    benc2 = b_enc.reshape(D + C_FEAT, 1)
    return pl.pallas_call(
        _enc_body,
        grid=(BN,),
        in_specs=[
            pl.BlockSpec((1, 3, H, W), lambda i: (i, 0, 0, 0)),
            pl.BlockSpec((D + C_FEAT, 3), lambda i: (0, 0)),
            pl.BlockSpec((D + C_FEAT, 1), lambda i: (0, 0)),
            pl.BlockSpec((1, 3, D, DSH, DSW), lambda i: (i, 0, 0, 0, 0)),
        ],
        out_specs=[
            pl.BlockSpec((1, D, DSH, DSW), lambda i: (i, 0, 0, 0)),
            pl.BlockSpec((1, C_FEAT, DSH, DSW), lambda i: (i, 0, 0, 0)),
            pl.BlockSpec((1, D, DSH, DSW), lambda i: (i, 0, 0, 0)),
            pl.BlockSpec((1, D, DSH, DSW), lambda i: (i, 0, 0, 0)),
        ],
        out_shape=[
            jax.ShapeDtypeStruct((BN, D, DSH, DSW), jnp.float32),
            jax.ShapeDtypeStruct((BN, C_FEAT, DSH, DSW), jnp.float32),
            jax.ShapeDtypeStruct((BN, D, DSH, DSW), jnp.float32),
            jax.ShapeDtypeStruct((BN, D, DSH, DSW), jnp.int32),
        ],
    )(img, W_enc, benc2, geom_t)


NTILE = 16
PIX_T = HW * N // NTILE
PTS_T_RAW = PIX_T * D
BLKP = 128
NBLK = (PTS_T_RAW + BLKP - 1) // BLKP
NPASS = 2
CH = C_FEAT // NPASS
ROWS_T = 2512
VROWS = NTILE * ROWS_T
OUT_ROWS = 320

ROWS_PTS = 128
PTS_PAD = ROWS_PTS * 128
OFF_RW = 0
OFF_PIX = OFF_RW + B * NTILE * ROWS_PTS
ROWS_FPT = 72
OFF_FLG = OFF_PIX + ROWS_PTS
OFF_FEAT = OFF_FLG + B * NTILE
BLOB_ROWS = OFF_FEAT + B * NPASS * NTILE * ROWS_FPT
W_DESCALE = 1.0 / (65535.0 * float(2 ** 20))


def _sc_scatter_body(blob_hbm, zeros_hbm, out_hbm,
                     feat_v, rw_v, pix_v, idx_v, rows_v, flags_v,
                     grid_sh):
    b = lax.axis_index("c")
    s = lax.axis_index("s")

    pltpu.sync_copy(blob_hbm.at[pl.ds(OFF_PIX, ROWS_PTS)], pix_v)
    pltpu.sync_copy(blob_hbm.at[pl.ds(OFF_FLG, B * NTILE)], flags_v)
    pltpu.sync_copy(blob_hbm.at[pl.ds(OFF_RW + (b * NTILE + s) * ROWS_PTS,
                                      ROWS_PTS)], rw_v)

    def pass_body(p, _):
        pltpu.sync_copy(zeros_hbm, grid_sh.at[pl.ds(s * ROWS_T, ROWS_T)])
        pltpu.sync_copy(
            blob_hbm.at[pl.ds(OFF_FEAT + ((b * NPASS + p) * NTILE + s)
                              * ROWS_FPT, ROWS_FPT)], feat_v)
        plsc.subcore_barrier()

        def blk_body(blk, _):
            def grp_body(g, _):
                q16 = g * 16
                pvec = rw_v[blk, pl.ds(q16, 16)]
                idx_v[0, pl.ds(q16, 16)] = lax.shift_right_logical(pvec, 16)
                pixvec = pix_v[blk, pl.ds(q16, 16)]
                for l in range(16):
                    wl = ((pvec[l] & 0xFFFF).astype(jnp.float32) * W_DESCALE)
                    pixl = pixvec[l]
                    fr = lax.shift_right_logical(pixl, 2)
                    fc = (pixl & 3) * CH
                    for j in range(CH // 16):
                        fvec = (feat_v[fr, pl.ds(fc + 16 * j, 16)]
                                .astype(jnp.float32))
                        rows_v[g * 16 + l, 16 * j:16 * (j + 1)] = wl * fvec
                return 0
            lax.fori_loop(0, BLKP // 16, grp_body, 0)
            pltpu.sync_copy(rows_v, grid_sh.at[idx_v.at[0]], add=True)
            return 0
        lax.fori_loop(0, NBLK, blk_body, 0)
        plsc.subcore_barrier()

        pltpu.sync_copy(grid_sh.at[pl.ds(s * ROWS_T, ROWS_T)],
                        out_hbm.at[b, p, pl.ds(s * ROWS_T, ROWS_T)])
        plsc.subcore_barrier()
        return 0

    lax.fori_loop(0, NPASS, pass_body, 0)


def _sc_scatter(blob):
    mesh = plsc.VectorSubcoreMesh(core_axis_name="c", subcore_axis_name="s")
    f = pl.kernel(
        _sc_scatter_body, mesh=mesh,
        compiler_params=pltpu.CompilerParams(use_tc_tiling_on_sc=False),
        out_type=jax.ShapeDtypeStruct((B, NPASS, VROWS, CH), jnp.float32),
        scratch_types=[
            pltpu.VMEM((ROWS_FPT, 128), jnp.int32),
            pltpu.VMEM((ROWS_PTS, 128), jnp.int32),
            pltpu.VMEM((ROWS_PTS, 128), jnp.int32),
            pltpu.VMEM((1, 128), jnp.int32),
            pltpu.VMEM((BLKP, CH), jnp.float32),
            pltpu.VMEM((B * NTILE, 128), jnp.int32),
            pltpu.VMEM_SHARED((VROWS, CH), jnp.float32),
        ],
    )
    return f(blob, jnp.zeros((ROWS_T, CH), jnp.float32))


def kernel(image, post_tran, post_rot, intrinsic, extrinsic, W_enc, b_enc):
    frustum = _frustum()
    ext_inv = jnp.linalg.inv(extrinsic)
    rotation = ext_inv[..., :3, :3]
    translation = ext_inv[..., :3, 3]
    points = frustum[None, None] - post_tran[:, :, None, None, None, :]
    points = jnp.einsum('bnij,bndhwj->bndhwi', jnp.linalg.inv(post_rot), points)
    points = jnp.concatenate((points[..., :2] * points[..., 2:3], points[..., 2:3]), axis=-1)
    combine = jnp.einsum('bnij,bnjk->bnik', rotation, jnp.linalg.inv(intrinsic))
    points = jnp.einsum('bnij,bndhwj->bndhwi', combine, points)
    geom = points + translation[:, :, None, None, None, :]

    geom_t = geom.reshape(BN, D, DSH, DSW, 3).transpose(0, 4, 1, 2, 3)

    depth, feat, w, rank = _encode(image, W_enc, b_enc, geom_t)

    depth_out = depth.reshape(B, N, D, DSH, DSW)

    npix = N * HW
    w_pm = w.reshape(B, N, D, HW).transpose(0, 1, 3, 2).reshape(B, npix * D)
    r_pm = rank.reshape(B, N, D, HW).transpose(0, 1, 3, 2).reshape(B, npix * D)
    w16 = jnp.round(w_pm * 65535.0).astype(jnp.uint32)
    packed = (r_pm.astype(jnp.uint32) << 16) | w16
    packed = jnp.pad(packed.reshape(B, NTILE, PTS_T_RAW),
                     ((0, 0), (0, 0), (0, PTS_PAD - PTS_T_RAW)),
                     constant_values=jnp.uint32((VROWS - 1) << 16))
    pix_map = jnp.minimum(jnp.arange(PTS_PAD, dtype=jnp.int32) // D, PIX_T - 1)
    feat_sc = (feat.reshape(B, N, C_FEAT, HW).transpose(0, 1, 3, 2)
               .reshape(B, npix, NPASS, CH).transpose(0, 2, 1, 3)
               .reshape(B, NPASS, NTILE, PIX_T * CH))
    feat_sc = jnp.pad(feat_sc,
                      ((0, 0), (0, 0), (0, 0), (0, ROWS_FPT * 128 - PIX_T * CH)))

    w16_blk = (packed & 0xFFFF).reshape(B, NTILE, ROWS_PTS, 128)
    flags = (w16_blk > 0).any(axis=-1).astype(jnp.int32)

    blob = jnp.concatenate([
        lax.bitcast_convert_type(packed, jnp.int32).reshape(-1),
        pix_map,
        flags.reshape(-1),
        (feat_sc * (2.0 ** 20)).astype(jnp.int32).reshape(-1),
    ]).reshape(BLOB_ROWS, 128)

    grid = _sc_scatter(blob)
    flat = grid.transpose(0, 2, 1, 3).reshape(B, VROWS, C_FEAT)[:, :NSEG]
    output = flat.reshape(B, X_DIM, Y_DIM, C_FEAT).transpose(0, 3, 1, 2)
    return output, depth_out
